# baseline (speedup 1.0000x reference)
import jax
import jax.numpy as jnp
from jax import lax
from jax.experimental import pallas as pl
from jax.experimental.pallas import tpu as pltpu
from jax.experimental.pallas import tpu_sc as plsc

E = 16
TOPK = 2
N = 16384
D = 2048
NK = N * TOPK
NC = 2
NS = 16
NW = NC * NS
C = NK // NW
RI = C // 128

RB = 16
NCH = C // RB
NSLOT = 3
LAG = 1


def _count_chunk(keys_ref):
    def jbody(j, accs):
        out = list(accs)
        for oi in range(8):
            kv = keys_ref[j, pl.ds(oi * 16, 16)]
            for e in range(E):
                out[e] = out[e] + jnp.where(kv == e, 1, 0)
        return tuple(out)

    accs = lax.fori_loop(
        0, RI, jbody, tuple(jnp.zeros((16,), jnp.int32) for _ in range(E)))
    lanes = lax.iota(jnp.int32, 16)
    cnt = jnp.zeros((16,), jnp.int32)
    for e in range(E):
        cnt = cnt + jnp.where(lanes == e, jnp.sum(accs[e]), 0)
    return cnt


def _dest_pass(keys_ref, offs, dest_ref, vals_ref, chunk_base):
    def jbody(j, carries):
        cs = list(carries)
        for oi in range(8):
            o = oi * 16
            kv = keys_ref[j, pl.ds(o, 16)]
            dest = jnp.zeros((16,), jnp.int32)
            for e in range(E):
                m = kv == e
                ci = plsc.cumsum(jnp.where(m, 1, 0))
                dest = dest + jnp.where(m, (cs[e] - 1) + ci, 0)
                cs[e] = cs[e] + ci[15]
            dest_ref[j, pl.ds(o, 16)] = dest
            vals_ref[j, pl.ds(o, 16)] = (
                chunk_base + j * 128 + o + lax.iota(jnp.int32, 16)) >> 1
        return tuple(cs)

    lax.fori_loop(0, RI, jbody, tuple(offs[e] for e in range(E)))


def _scatter_chunk(dest_ref, vals_ref, sc_ref, tok_sh, ss_sh, sem):
    copies = []
    for j in range(RI):
        copies.append(pltpu.async_copy(
            vals_ref.at[j], tok_sh.at[dest_ref.at[j]], sem))
        copies.append(pltpu.async_copy(
            sc_ref.at[j], ss_sh.at[dest_ref.at[j]], sem))
    for cp in copies:
        cp.wait()


def _fused_body(keys_hbm, scores_hbm, x_hbm,
                counts_hbm, tokidx_hbm, ssort_hbm, out_hbm,
                keys_v, keys2_v, dest_v, vals_v, sc_v, cnt_v, cnt2_v,
                offs_v, total_v, grid_v, grid_hbm,
                guard_sh, tok_sh, ss_sh,
                idx_v, sv_v, rows0, rows1, rows2,
                sem, osem, g0, g1, g2, p0, p1, p2):
    c = lax.axis_index("c")
    s = lax.axis_index("s")
    wid = c * NS + s
    base = wid * C

    oc_wid = (1 - c) * NS + s
    pltpu.sync_copy(keys_hbm.at[pl.ds(oc_wid * RI, RI), :], keys2_v)
    cnt2_v[...] = _count_chunk(keys2_v)

    pltpu.sync_copy(keys_hbm.at[pl.ds(wid * RI, RI), :], keys_v)
    cnt_v[...] = _count_chunk(keys_v)

    pltpu.sync_copy(cnt_v, grid_hbm.at[c, wid])
    pltpu.sync_copy(cnt2_v, grid_hbm.at[c, oc_wid])
    plsc.subcore_barrier()
    pltpu.sync_copy(grid_hbm.at[c], grid_v)

    total_v[...] = jnp.zeros((16,), jnp.int32)
    offs_v[...] = jnp.zeros((16,), jnp.int32)
    cnt2_v[...] = jnp.zeros((16,), jnp.int32)

    def wbody(w, _):
        row = grid_v[w, pl.ds(0, 16)]
        total_v[...] = total_v[...] + row
        offs_v[...] = offs_v[...] + jnp.where(w < wid, row, 0)
        cnt2_v[...] = cnt2_v[...] + jnp.where(w < oc_wid, row, 0)
        return 0

    lax.fori_loop(0, NW, wbody, 0)
    tot = total_v[...]
    excl = plsc.cumsum(tot) - tot
    offs_v[...] = offs_v[...] + excl
    cnt2_v[...] = cnt2_v[...] + excl

    @pl.when(wid == 0)
    def _():
        pltpu.sync_copy(total_v, counts_hbm)

    _dest_pass(keys_v, offs_v[...], dest_v, vals_v, base)
    pltpu.sync_copy(scores_hbm.at[pl.ds(wid * RI, RI), :], sc_v)
    _scatter_chunk(dest_v, vals_v, sc_v, tok_sh, ss_sh, sem)

    _dest_pass(keys2_v, cnt2_v[...], dest_v, vals_v, oc_wid * C)
    pltpu.sync_copy(scores_hbm.at[pl.ds(oc_wid * RI, RI), :], sc_v)
    _scatter_chunk(dest_v, vals_v, sc_v, tok_sh, ss_sh, sem)

    plsc.subcore_barrier()

    pltpu.sync_copy(tok_sh.at[pl.ds(base, C)], idx_v)
    pltpu.sync_copy(ss_sh.at[pl.ds(base, C)], sv_v)
    out1 = pltpu.async_copy(idx_v, tokidx_hbm.at[pl.ds(base, C)], osem)
    out2 = pltpu.async_copy(sv_v, ssort_hbm.at[pl.ds(base, C)], osem)

    bufs = (rows0, rows1, rows2)
    gsems = (g0, g1, g2)
    psems = (p0, p1, p2)

    def fetch(ch, slot):
        return pltpu.async_copy(
            x_hbm.at[idx_v.at[pl.ds(ch * RB, RB)]], bufs[slot], gsems[slot])

    def drain_put(slot):
        pltpu.make_async_copy(
            bufs[slot], out_hbm.at[pl.ds(0, RB), :], psems[slot]).wait()

    def drain_get(slot):
        pltpu.make_async_copy(
            x_hbm.at[pl.ds(0, RB), :], bufs[slot], gsems[slot]).wait()

    def process(ch, b):
        drain_get(b)
        buf = bufs[b]
        sbs = [plsc.load_gather(
                   sv_v, [jnp.full((16,), ch * RB + r, jnp.int32)])
               for r in range(RB)]

        def dbody(d, _):
            o = d * 16
            for r in range(RB):
                buf[r, pl.ds(o, 16)] = buf[r, pl.ds(o, 16)] * sbs[r]
            return 0

        lax.fori_loop(0, D // 16, dbody, 0)
        pltpu.async_copy(
            buf, out_hbm.at[pl.ds(base + ch * RB, RB), :], psems[b])

    for slot in range(NSLOT):
        fetch(slot, slot)

    nround = (NCH - 1) // NSLOT

    def round_body(g, _):
        for b in range(NSLOT):
            ch = g * NSLOT + b
            process(ch, b)
            fslot = (b + LAG) % NSLOT

            @pl.when(jnp.logical_and(ch >= NSLOT - LAG, ch + LAG < NCH))
            def _():
                drain_put(fslot)
                fetch(ch + LAG, fslot)

        return 0

    lax.fori_loop(0, nround, round_body, 0)
    for ch in range(nround * NSLOT, NCH):
        process(ch, ch % NSLOT)
    for slot in range(NSLOT):
        drain_put(slot)
    out1.wait()
    out2.wait()


@jax.jit
def kernel(x, top_scores, selected_experts_indices, num_tokens_per_expert):
    del num_tokens_per_expert
    keys2 = selected_experts_indices.astype(jnp.int32).reshape(NK // 128, 128)
    scores2 = top_scores.astype(jnp.float32).reshape(NK // 128, 128)

    mesh = plsc.VectorSubcoreMesh(core_axis_name="c", subcore_axis_name="s",
                                  num_cores=NC, num_subcores=NS)

    fused = pl.kernel(
        _fused_body,
        compiler_params=pltpu.CompilerParams(needs_layout_passes=False),
        out_type=(
            jax.ShapeDtypeStruct((E,), jnp.int32),
            jax.ShapeDtypeStruct((NK,), jnp.int32),
            jax.ShapeDtypeStruct((NK,), jnp.float32),
            jax.ShapeDtypeStruct((NK, D), jnp.float32),
        ),
        mesh=mesh,
        scratch_types=[
            pltpu.VMEM((RI, 128), jnp.int32),
            pltpu.VMEM((RI, 128), jnp.int32),
            pltpu.VMEM((RI, 128), jnp.int32),
            pltpu.VMEM((RI, 128), jnp.int32),
            pltpu.VMEM((RI, 128), jnp.float32),
            pltpu.VMEM((16,), jnp.int32),
            pltpu.VMEM((16,), jnp.int32),
            pltpu.VMEM((16,), jnp.int32),
            pltpu.VMEM((16,), jnp.int32),
            pltpu.VMEM((NW, 16), jnp.int32),
            pltpu.HBM((NC, NW, 16), jnp.int32),
            pltpu.VMEM_SHARED((256,), jnp.int32),
            pltpu.VMEM_SHARED((NK,), jnp.int32),
            pltpu.VMEM_SHARED((NK,), jnp.float32),
            pltpu.VMEM((C,), jnp.int32),
            pltpu.VMEM((C,), jnp.float32),
            pltpu.VMEM((RB, D), jnp.float32),
            pltpu.VMEM((RB, D), jnp.float32),
            pltpu.VMEM((RB, D), jnp.float32),
            pltpu.SemaphoreType.DMA,
            pltpu.SemaphoreType.DMA,
            pltpu.SemaphoreType.DMA,
            pltpu.SemaphoreType.DMA,
            pltpu.SemaphoreType.DMA,
            pltpu.SemaphoreType.DMA,
            pltpu.SemaphoreType.DMA,
            pltpu.SemaphoreType.DMA,
        ],
    )
    counts, tokidx, ssort, routed = fused(keys2, scores2, x)
    return routed, counts, tokidx, ssort


# speedup vs baseline: 1.3191x; 1.3191x over previous
import jax
import jax.numpy as jnp
from jax import lax
from jax.experimental import pallas as pl
from jax.experimental.pallas import tpu as pltpu
from jax.experimental.pallas import tpu_sc as plsc

E = 16
TOPK = 2
N = 16384
D = 2048
NK = N * TOPK
NC = 2
NS = 16
NW = NC * NS
C = NK // NW
RI = C // 128

RB = 8
NCH = C // RB
NSLOT = 4
LAG = 2


def _count_chunk(keys_ref):
    def jbody(j, accs):
        out = list(accs)
        for oi in range(8):
            kv = keys_ref[j, pl.ds(oi * 16, 16)]
            for e in range(E):
                out[e] = out[e] + jnp.where(kv == e, 1, 0)
        return tuple(out)

    accs = lax.fori_loop(
        0, RI, jbody, tuple(jnp.zeros((16,), jnp.int32) for _ in range(E)))
    lanes = lax.iota(jnp.int32, 16)
    cnt = jnp.zeros((16,), jnp.int32)
    for e in range(E):
        cnt = cnt + jnp.where(lanes == e, jnp.sum(accs[e]), 0)
    return cnt


def _dest_pass(keys_ref, offs, dest_ref, vals_ref, chunk_base):
    def jbody(j, carries):
        cs = list(carries)
        for oi in range(8):
            o = oi * 16
            kv = keys_ref[j, pl.ds(o, 16)]
            dest = jnp.zeros((16,), jnp.int32)
            for e in range(E):
                m = kv == e
                ci = plsc.cumsum(jnp.where(m, 1, 0))
                dest = dest + jnp.where(m, (cs[e] - 1) + ci, 0)
                cs[e] = cs[e] + ci[15]
            dest_ref[j, pl.ds(o, 16)] = dest
            vals_ref[j, pl.ds(o, 16)] = (
                chunk_base + j * 128 + o + lax.iota(jnp.int32, 16)) >> 1
        return tuple(cs)

    lax.fori_loop(0, RI, jbody, tuple(offs[e] for e in range(E)))


def _scatter_chunk(dest_ref, vals_ref, sc_ref, tok_sh, ss_sh, sem):
    copies = []
    for j in range(RI):
        copies.append(pltpu.async_copy(
            vals_ref.at[j], tok_sh.at[dest_ref.at[j]], sem))
        copies.append(pltpu.async_copy(
            sc_ref.at[j], ss_sh.at[dest_ref.at[j]], sem))
    for cp in copies:
        cp.wait()


def _fused_body(keys_hbm, scores_hbm, x_hbm,
                counts_hbm, tokidx_hbm, ssort_hbm, out_hbm,
                keys_v, keys2_v, dest_v, vals_v, sc_v, cnt_v, cnt2_v,
                offs_v, total_v, grid_v, grid_hbm,
                guard_sh, tok_sh, ss_sh,
                idx_v, sv_v, rows0, rows1, rows2, rows3,
                sem, osem, g0, g1, g2, g3, p0, p1, p2, p3):
    c = lax.axis_index("c")
    s = lax.axis_index("s")
    wid = c * NS + s
    base = wid * C

    oc_wid = (1 - c) * NS + s
    pltpu.sync_copy(keys_hbm.at[pl.ds(oc_wid * RI, RI), :], keys2_v)
    cnt2_v[...] = _count_chunk(keys2_v)

    pltpu.sync_copy(keys_hbm.at[pl.ds(wid * RI, RI), :], keys_v)
    cnt_v[...] = _count_chunk(keys_v)

    pltpu.sync_copy(cnt_v, grid_hbm.at[c, wid])
    pltpu.sync_copy(cnt2_v, grid_hbm.at[c, oc_wid])
    plsc.subcore_barrier()
    pltpu.sync_copy(grid_hbm.at[c], grid_v)

    total_v[...] = jnp.zeros((16,), jnp.int32)
    offs_v[...] = jnp.zeros((16,), jnp.int32)
    cnt2_v[...] = jnp.zeros((16,), jnp.int32)

    def wbody(w, _):
        row = grid_v[w, pl.ds(0, 16)]
        total_v[...] = total_v[...] + row
        offs_v[...] = offs_v[...] + jnp.where(w < wid, row, 0)
        cnt2_v[...] = cnt2_v[...] + jnp.where(w < oc_wid, row, 0)
        return 0

    lax.fori_loop(0, NW, wbody, 0)
    tot = total_v[...]
    excl = plsc.cumsum(tot) - tot
    offs_v[...] = offs_v[...] + excl
    cnt2_v[...] = cnt2_v[...] + excl

    @pl.when(wid == 0)
    def _():
        pltpu.sync_copy(total_v, counts_hbm)

    _dest_pass(keys_v, offs_v[...], dest_v, vals_v, base)
    pltpu.sync_copy(scores_hbm.at[pl.ds(wid * RI, RI), :], sc_v)
    _scatter_chunk(dest_v, vals_v, sc_v, tok_sh, ss_sh, sem)

    _dest_pass(keys2_v, cnt2_v[...], dest_v, vals_v, oc_wid * C)
    pltpu.sync_copy(scores_hbm.at[pl.ds(oc_wid * RI, RI), :], sc_v)
    _scatter_chunk(dest_v, vals_v, sc_v, tok_sh, ss_sh, sem)

    plsc.subcore_barrier()

    pltpu.sync_copy(tok_sh.at[pl.ds(base, C)], idx_v)
    pltpu.sync_copy(ss_sh.at[pl.ds(base, C)], sv_v)
    out1 = pltpu.async_copy(idx_v, tokidx_hbm.at[pl.ds(base, C)], osem)
    out2 = pltpu.async_copy(sv_v, ssort_hbm.at[pl.ds(base, C)], osem)

    bufs = (rows0, rows1, rows2, rows3)
    gsems = (g0, g1, g2, g3)
    psems = (p0, p1, p2, p3)

    def fetch(ch, slot):
        return pltpu.async_copy(
            x_hbm.at[idx_v.at[pl.ds(ch * RB, RB)]], bufs[slot], gsems[slot])

    def drain_put(slot):
        pltpu.make_async_copy(
            bufs[slot], out_hbm.at[pl.ds(0, RB), :], psems[slot]).wait()

    def drain_get(slot):
        pltpu.make_async_copy(
            x_hbm.at[pl.ds(0, RB), :], bufs[slot], gsems[slot]).wait()

    def process(ch, b):
        drain_get(b)
        buf = bufs[b]
        sbs = [plsc.load_gather(
                   sv_v, [jnp.full((16,), ch * RB + r, jnp.int32)])
               for r in range(RB)]

        def dbody(d, _):
            o = d * 16
            for r in range(RB):
                buf[r, pl.ds(o, 16)] = buf[r, pl.ds(o, 16)] * sbs[r]
            return 0

        lax.fori_loop(0, D // 16, dbody, 0)
        pltpu.async_copy(
            buf, out_hbm.at[pl.ds(base + ch * RB, RB), :], psems[b])

    for slot in range(NSLOT):
        fetch(slot, slot)

    nround = NCH // NSLOT
    assert NCH - nround * NSLOT <= LAG

    def round_body(g, _):
        for b in range(NSLOT):
            ch = g * NSLOT + b
            process(ch, b)
            fslot = (b + LAG) % NSLOT

            @pl.when(jnp.logical_and(ch >= NSLOT - LAG, ch + LAG < NCH))
            def _():
                drain_put(fslot)
                fetch(ch + LAG, fslot)

        return 0

    lax.fori_loop(0, nround, round_body, 0)
    for ch in range(nround * NSLOT, NCH):
        process(ch, ch % NSLOT)
    for slot in range(NSLOT):
        drain_put(slot)
    out1.wait()
    out2.wait()


@jax.jit
def kernel(x, top_scores, selected_experts_indices, num_tokens_per_expert):
    del num_tokens_per_expert
    keys2 = selected_experts_indices.astype(jnp.int32).reshape(NK // 128, 128)
    scores2 = top_scores.astype(jnp.float32).reshape(NK // 128, 128)

    mesh = plsc.VectorSubcoreMesh(core_axis_name="c", subcore_axis_name="s",
                                  num_cores=NC, num_subcores=NS)

    fused = pl.kernel(
        _fused_body,
        compiler_params=pltpu.CompilerParams(needs_layout_passes=False),
        out_type=(
            jax.ShapeDtypeStruct((E,), jnp.int32),
            jax.ShapeDtypeStruct((NK,), jnp.int32),
            jax.ShapeDtypeStruct((NK,), jnp.float32),
            jax.ShapeDtypeStruct((NK, D), jnp.float32),
        ),
        mesh=mesh,
        scratch_types=[
            pltpu.VMEM((RI, 128), jnp.int32),
            pltpu.VMEM((RI, 128), jnp.int32),
            pltpu.VMEM((RI, 128), jnp.int32),
            pltpu.VMEM((RI, 128), jnp.int32),
            pltpu.VMEM((RI, 128), jnp.float32),
            pltpu.VMEM((16,), jnp.int32),
            pltpu.VMEM((16,), jnp.int32),
            pltpu.VMEM((16,), jnp.int32),
            pltpu.VMEM((16,), jnp.int32),
            pltpu.VMEM((NW, 16), jnp.int32),
            pltpu.HBM((NC, NW, 16), jnp.int32),
            pltpu.VMEM_SHARED((256,), jnp.int32),
            pltpu.VMEM_SHARED((NK,), jnp.int32),
            pltpu.VMEM_SHARED((NK,), jnp.float32),
            pltpu.VMEM((C,), jnp.int32),
            pltpu.VMEM((C,), jnp.float32),
            pltpu.VMEM((RB, D), jnp.float32),
            pltpu.VMEM((RB, D), jnp.float32),
            pltpu.VMEM((RB, D), jnp.float32),
            pltpu.VMEM((RB, D), jnp.float32),
            pltpu.SemaphoreType.DMA,
            pltpu.SemaphoreType.DMA,
            pltpu.SemaphoreType.DMA,
            pltpu.SemaphoreType.DMA,
            pltpu.SemaphoreType.DMA,
            pltpu.SemaphoreType.DMA,
            pltpu.SemaphoreType.DMA,
            pltpu.SemaphoreType.DMA,
            pltpu.SemaphoreType.DMA,
            pltpu.SemaphoreType.DMA,
        ],
    )
    counts, tokidx, ssort, routed = fused(keys2, scores2, x)
    return routed, counts, tokidx, ssort


# speedup vs baseline: 1.3766x; 1.0435x over previous
import jax
import jax.numpy as jnp
from jax import lax
from jax.experimental import pallas as pl
from jax.experimental.pallas import tpu as pltpu
from jax.experimental.pallas import tpu_sc as plsc

E = 16
TOPK = 2
N = 16384
D = 2048
NK = N * TOPK
NC = 2
NS = 16
NW = NC * NS
C = NK // NW
RI = C // 128

RB = 8
NCH = C // RB
NSLOT = 6
LAG = 3


def _count_chunk(keys_ref):
    def jbody(j, accs):
        out = list(accs)
        for oi in range(8):
            kv = keys_ref[j, pl.ds(oi * 16, 16)]
            for e in range(E):
                out[e] = out[e] + jnp.where(kv == e, 1, 0)
        return tuple(out)

    accs = lax.fori_loop(
        0, RI, jbody, tuple(jnp.zeros((16,), jnp.int32) for _ in range(E)))
    lanes = lax.iota(jnp.int32, 16)
    cnt = jnp.zeros((16,), jnp.int32)
    for e in range(E):
        cnt = cnt + jnp.where(lanes == e, jnp.sum(accs[e]), 0)
    return cnt


def _dest_pass(keys_ref, offs, dest_ref, vals_ref, chunk_base):
    def jbody(j, carries):
        cs = list(carries)
        for oi in range(8):
            o = oi * 16
            kv = keys_ref[j, pl.ds(o, 16)]
            dest = jnp.zeros((16,), jnp.int32)
            for e in range(E):
                m = kv == e
                ci = plsc.cumsum(jnp.where(m, 1, 0))
                dest = dest + jnp.where(m, (cs[e] - 1) + ci, 0)
                cs[e] = cs[e] + ci[15]
            dest_ref[j, pl.ds(o, 16)] = dest
            vals_ref[j, pl.ds(o, 16)] = (
                chunk_base + j * 128 + o + lax.iota(jnp.int32, 16)) >> 1
        return tuple(cs)

    lax.fori_loop(0, RI, jbody, tuple(offs[e] for e in range(E)))


def _scatter_chunk(dest_ref, vals_ref, sc_ref, tok_sh, ss_sh, sem):
    copies = []
    for j in range(RI):
        copies.append(pltpu.async_copy(
            vals_ref.at[j], tok_sh.at[dest_ref.at[j]], sem))
        copies.append(pltpu.async_copy(
            sc_ref.at[j], ss_sh.at[dest_ref.at[j]], sem))
    for cp in copies:
        cp.wait()


def _fused_body(keys_hbm, scores_hbm, x_hbm,
                counts_hbm, tokidx_hbm, ssort_hbm, out_hbm,
                keys_v, keys2_v, dest_v, vals_v, sc_v, cnt_v, cnt2_v,
                offs_v, total_v, grid_v, grid_hbm,
                guard_sh, tok_sh, ss_sh,
                idx_v, sv_v, rows0, rows1, rows2, rows3, rows4, rows5,
                sem, osem, g0, g1, g2, g3, g4, g5, p0, p1, p2, p3, p4, p5):
    c = lax.axis_index("c")
    s = lax.axis_index("s")
    wid = c * NS + s
    base = wid * C

    oc_wid = (1 - c) * NS + s
    pltpu.sync_copy(keys_hbm.at[pl.ds(oc_wid * RI, RI), :], keys2_v)
    cnt2_v[...] = _count_chunk(keys2_v)

    pltpu.sync_copy(keys_hbm.at[pl.ds(wid * RI, RI), :], keys_v)
    cnt_v[...] = _count_chunk(keys_v)

    pltpu.sync_copy(cnt_v, grid_hbm.at[c, wid])
    pltpu.sync_copy(cnt2_v, grid_hbm.at[c, oc_wid])
    plsc.subcore_barrier()
    pltpu.sync_copy(grid_hbm.at[c], grid_v)

    total_v[...] = jnp.zeros((16,), jnp.int32)
    offs_v[...] = jnp.zeros((16,), jnp.int32)
    cnt2_v[...] = jnp.zeros((16,), jnp.int32)

    def wbody(w, _):
        row = grid_v[w, pl.ds(0, 16)]
        total_v[...] = total_v[...] + row
        offs_v[...] = offs_v[...] + jnp.where(w < wid, row, 0)
        cnt2_v[...] = cnt2_v[...] + jnp.where(w < oc_wid, row, 0)
        return 0

    lax.fori_loop(0, NW, wbody, 0)
    tot = total_v[...]
    excl = plsc.cumsum(tot) - tot
    offs_v[...] = offs_v[...] + excl
    cnt2_v[...] = cnt2_v[...] + excl

    @pl.when(wid == 0)
    def _():
        pltpu.sync_copy(total_v, counts_hbm)

    _dest_pass(keys_v, offs_v[...], dest_v, vals_v, base)
    pltpu.sync_copy(scores_hbm.at[pl.ds(wid * RI, RI), :], sc_v)
    _scatter_chunk(dest_v, vals_v, sc_v, tok_sh, ss_sh, sem)

    _dest_pass(keys2_v, cnt2_v[...], dest_v, vals_v, oc_wid * C)
    pltpu.sync_copy(scores_hbm.at[pl.ds(oc_wid * RI, RI), :], sc_v)
    _scatter_chunk(dest_v, vals_v, sc_v, tok_sh, ss_sh, sem)

    plsc.subcore_barrier()

    pltpu.sync_copy(tok_sh.at[pl.ds(base, C)], idx_v)
    pltpu.sync_copy(ss_sh.at[pl.ds(base, C)], sv_v)
    out1 = pltpu.async_copy(idx_v, tokidx_hbm.at[pl.ds(base, C)], osem)
    out2 = pltpu.async_copy(sv_v, ssort_hbm.at[pl.ds(base, C)], osem)

    bufs = (rows0, rows1, rows2, rows3, rows4, rows5)
    gsems = (g0, g1, g2, g3, g4, g5)
    psems = (p0, p1, p2, p3, p4, p5)

    def fetch(ch, slot):
        return pltpu.async_copy(
            x_hbm.at[idx_v.at[pl.ds(ch * RB, RB)]], bufs[slot], gsems[slot])

    def drain_put(slot):
        pltpu.make_async_copy(
            bufs[slot], out_hbm.at[pl.ds(0, RB), :], psems[slot]).wait()

    def drain_get(slot):
        pltpu.make_async_copy(
            x_hbm.at[pl.ds(0, RB), :], bufs[slot], gsems[slot]).wait()

    def process(ch, b):
        drain_get(b)
        buf = bufs[b]
        sbs = [plsc.load_gather(
                   sv_v, [jnp.full((16,), ch * RB + r, jnp.int32)])
               for r in range(RB)]

        def dbody(d, _):
            o = d * 16
            for r in range(RB):
                buf[r, pl.ds(o, 16)] = buf[r, pl.ds(o, 16)] * sbs[r]
            return 0

        lax.fori_loop(0, D // 16, dbody, 0)
        pltpu.async_copy(
            buf, out_hbm.at[pl.ds(base + ch * RB, RB), :], psems[b])

    for slot in range(NSLOT):
        fetch(slot, slot)

    nround = NCH // NSLOT
    assert NCH - nround * NSLOT <= LAG

    def round_body(g, _):
        for b in range(NSLOT):
            ch = g * NSLOT + b
            process(ch, b)
            fslot = (b + LAG) % NSLOT

            @pl.when(jnp.logical_and(ch >= NSLOT - LAG, ch + LAG < NCH))
            def _():
                drain_put(fslot)
                fetch(ch + LAG, fslot)

        return 0

    lax.fori_loop(0, nround, round_body, 0)
    for ch in range(nround * NSLOT, NCH):
        process(ch, ch % NSLOT)
    for slot in range(NSLOT):
        drain_put(slot)
    out1.wait()
    out2.wait()


@jax.jit
def kernel(x, top_scores, selected_experts_indices, num_tokens_per_expert):
    del num_tokens_per_expert
    keys2 = selected_experts_indices.astype(jnp.int32).reshape(NK // 128, 128)
    scores2 = top_scores.astype(jnp.float32).reshape(NK // 128, 128)

    mesh = plsc.VectorSubcoreMesh(core_axis_name="c", subcore_axis_name="s",
                                  num_cores=NC, num_subcores=NS)

    fused = pl.kernel(
        _fused_body,
        compiler_params=pltpu.CompilerParams(needs_layout_passes=False),
        out_type=(
            jax.ShapeDtypeStruct((E,), jnp.int32),
            jax.ShapeDtypeStruct((NK,), jnp.int32),
            jax.ShapeDtypeStruct((NK,), jnp.float32),
            jax.ShapeDtypeStruct((NK, D), jnp.float32),
        ),
        mesh=mesh,
        scratch_types=[
            pltpu.VMEM((RI, 128), jnp.int32),
            pltpu.VMEM((RI, 128), jnp.int32),
            pltpu.VMEM((RI, 128), jnp.int32),
            pltpu.VMEM((RI, 128), jnp.int32),
            pltpu.VMEM((RI, 128), jnp.float32),
            pltpu.VMEM((16,), jnp.int32),
            pltpu.VMEM((16,), jnp.int32),
            pltpu.VMEM((16,), jnp.int32),
            pltpu.VMEM((16,), jnp.int32),
            pltpu.VMEM((NW, 16), jnp.int32),
            pltpu.HBM((NC, NW, 16), jnp.int32),
            pltpu.VMEM_SHARED((256,), jnp.int32),
            pltpu.VMEM_SHARED((NK,), jnp.int32),
            pltpu.VMEM_SHARED((NK,), jnp.float32),
            pltpu.VMEM((C,), jnp.int32),
            pltpu.VMEM((C,), jnp.float32),
            pltpu.VMEM((RB, D), jnp.float32),
            pltpu.VMEM((RB, D), jnp.float32),
            pltpu.VMEM((RB, D), jnp.float32),
            pltpu.VMEM((RB, D), jnp.float32),
            pltpu.VMEM((RB, D), jnp.float32),
            pltpu.VMEM((RB, D), jnp.float32),
            pltpu.SemaphoreType.DMA,
            pltpu.SemaphoreType.DMA,
            pltpu.SemaphoreType.DMA,
            pltpu.SemaphoreType.DMA,
            pltpu.SemaphoreType.DMA,
            pltpu.SemaphoreType.DMA,
            pltpu.SemaphoreType.DMA,
            pltpu.SemaphoreType.DMA,
            pltpu.SemaphoreType.DMA,
            pltpu.SemaphoreType.DMA,
            pltpu.SemaphoreType.DMA,
            pltpu.SemaphoreType.DMA,
            pltpu.SemaphoreType.DMA,
            pltpu.SemaphoreType.DMA,
        ],
    )
    counts, tokidx, ssort, routed = fused(keys2, scores2, x)
    return routed, counts, tokidx, ssort


# speedup vs baseline: 1.3791x; 1.0019x over previous
import jax
import jax.numpy as jnp
from jax import lax
from jax.experimental import pallas as pl
from jax.experimental.pallas import tpu as pltpu
from jax.experimental.pallas import tpu_sc as plsc

E = 16
TOPK = 2
N = 16384
D = 2048
NK = N * TOPK
NC = 2
NS = 16
NW = NC * NS
C = NK // NW
RI = C // 128

RB = 8
NCH = C // RB
NSLOT = 6
LAG = 4


def _count_chunk(keys_ref):
    def jbody(j, accs):
        out = list(accs)
        for oi in range(8):
            kv = keys_ref[j, pl.ds(oi * 16, 16)]
            for e in range(E):
                out[e] = out[e] + jnp.where(kv == e, 1, 0)
        return tuple(out)

    accs = lax.fori_loop(
        0, RI, jbody, tuple(jnp.zeros((16,), jnp.int32) for _ in range(E)))
    lanes = lax.iota(jnp.int32, 16)
    cnt = jnp.zeros((16,), jnp.int32)
    for e in range(E):
        cnt = cnt + jnp.where(lanes == e, jnp.sum(accs[e]), 0)
    return cnt


def _dest_pass(keys_ref, offs, dest_ref, vals_ref, chunk_base):
    def jbody(j, carries):
        cs = list(carries)
        for oi in range(8):
            o = oi * 16
            kv = keys_ref[j, pl.ds(o, 16)]
            dest = jnp.zeros((16,), jnp.int32)
            for e in range(E):
                m = kv == e
                ci = plsc.cumsum(jnp.where(m, 1, 0))
                dest = dest + jnp.where(m, (cs[e] - 1) + ci, 0)
                cs[e] = cs[e] + ci[15]
            dest_ref[j, pl.ds(o, 16)] = dest
            vals_ref[j, pl.ds(o, 16)] = (
                chunk_base + j * 128 + o + lax.iota(jnp.int32, 16)) >> 1
        return tuple(cs)

    lax.fori_loop(0, RI, jbody, tuple(offs[e] for e in range(E)))


def _scatter_chunk(dest_ref, vals_ref, sc_ref, tok_sh, ss_sh, sem):
    copies = []
    for j in range(RI):
        copies.append(pltpu.async_copy(
            vals_ref.at[j], tok_sh.at[dest_ref.at[j]], sem))
        copies.append(pltpu.async_copy(
            sc_ref.at[j], ss_sh.at[dest_ref.at[j]], sem))
    for cp in copies:
        cp.wait()


def _fused_body(keys_hbm, scores_hbm, x_hbm,
                counts_hbm, tokidx_hbm, ssort_hbm, out_hbm,
                keys_v, keys2_v, dest_v, vals_v, sc_v, cnt_v, cnt2_v,
                offs_v, total_v, grid_v, grid_hbm,
                guard_sh, tok_sh, ss_sh,
                idx_v, sv_v, rows0, rows1, rows2, rows3, rows4, rows5,
                sem, osem, g0, g1, g2, g3, g4, g5, p0, p1, p2, p3, p4, p5):
    c = lax.axis_index("c")
    s = lax.axis_index("s")
    wid = c * NS + s
    base = wid * C

    oc_wid = (1 - c) * NS + s
    pltpu.sync_copy(keys_hbm.at[pl.ds(oc_wid * RI, RI), :], keys2_v)
    cnt2_v[...] = _count_chunk(keys2_v)

    pltpu.sync_copy(keys_hbm.at[pl.ds(wid * RI, RI), :], keys_v)
    cnt_v[...] = _count_chunk(keys_v)

    pltpu.sync_copy(cnt_v, grid_hbm.at[c, wid])
    pltpu.sync_copy(cnt2_v, grid_hbm.at[c, oc_wid])
    plsc.subcore_barrier()
    pltpu.sync_copy(grid_hbm.at[c], grid_v)

    total_v[...] = jnp.zeros((16,), jnp.int32)
    offs_v[...] = jnp.zeros((16,), jnp.int32)
    cnt2_v[...] = jnp.zeros((16,), jnp.int32)

    def wbody(w, _):
        row = grid_v[w, pl.ds(0, 16)]
        total_v[...] = total_v[...] + row
        offs_v[...] = offs_v[...] + jnp.where(w < wid, row, 0)
        cnt2_v[...] = cnt2_v[...] + jnp.where(w < oc_wid, row, 0)
        return 0

    lax.fori_loop(0, NW, wbody, 0)
    tot = total_v[...]
    excl = plsc.cumsum(tot) - tot
    offs_v[...] = offs_v[...] + excl
    cnt2_v[...] = cnt2_v[...] + excl

    @pl.when(wid == 0)
    def _():
        pltpu.sync_copy(total_v, counts_hbm)

    _dest_pass(keys_v, offs_v[...], dest_v, vals_v, base)
    pltpu.sync_copy(scores_hbm.at[pl.ds(wid * RI, RI), :], sc_v)
    _scatter_chunk(dest_v, vals_v, sc_v, tok_sh, ss_sh, sem)

    _dest_pass(keys2_v, cnt2_v[...], dest_v, vals_v, oc_wid * C)
    pltpu.sync_copy(scores_hbm.at[pl.ds(oc_wid * RI, RI), :], sc_v)
    _scatter_chunk(dest_v, vals_v, sc_v, tok_sh, ss_sh, sem)

    plsc.subcore_barrier()

    pltpu.sync_copy(tok_sh.at[pl.ds(base, C)], idx_v)
    pltpu.sync_copy(ss_sh.at[pl.ds(base, C)], sv_v)
    out1 = pltpu.async_copy(idx_v, tokidx_hbm.at[pl.ds(base, C)], osem)
    out2 = pltpu.async_copy(sv_v, ssort_hbm.at[pl.ds(base, C)], osem)

    bufs = (rows0, rows1, rows2, rows3, rows4, rows5)
    gsems = (g0, g1, g2, g3, g4, g5)
    psems = (p0, p1, p2, p3, p4, p5)

    def fetch(ch, slot):
        return pltpu.async_copy(
            x_hbm.at[idx_v.at[pl.ds(ch * RB, RB)]], bufs[slot], gsems[slot])

    def drain_put(slot):
        pltpu.make_async_copy(
            bufs[slot], out_hbm.at[pl.ds(0, RB), :], psems[slot]).wait()

    def drain_get(slot):
        pltpu.make_async_copy(
            x_hbm.at[pl.ds(0, RB), :], bufs[slot], gsems[slot]).wait()

    def process(ch, b):
        drain_get(b)
        buf = bufs[b]
        sbs = [plsc.load_gather(
                   sv_v, [jnp.full((16,), ch * RB + r, jnp.int32)])
               for r in range(RB)]

        def dbody(d, _):
            o = d * 16
            for r in range(RB):
                buf[r, pl.ds(o, 16)] = buf[r, pl.ds(o, 16)] * sbs[r]
            return 0

        lax.fori_loop(0, D // 16, dbody, 0)
        pltpu.async_copy(
            buf, out_hbm.at[pl.ds(base + ch * RB, RB), :], psems[b])

    for slot in range(NSLOT):
        fetch(slot, slot)

    nround = NCH // NSLOT
    assert NCH - nround * NSLOT <= LAG

    def round_body(g, _):
        for b in range(NSLOT):
            ch = g * NSLOT + b
            process(ch, b)
            fslot = (b + LAG) % NSLOT

            @pl.when(jnp.logical_and(ch >= NSLOT - LAG, ch + LAG < NCH))
            def _():
                drain_put(fslot)
                fetch(ch + LAG, fslot)

        return 0

    lax.fori_loop(0, nround, round_body, 0)
    for ch in range(nround * NSLOT, NCH):
        process(ch, ch % NSLOT)
    for slot in range(NSLOT):
        drain_put(slot)
    out1.wait()
    out2.wait()


@jax.jit
def kernel(x, top_scores, selected_experts_indices, num_tokens_per_expert):
    del num_tokens_per_expert
    keys2 = selected_experts_indices.astype(jnp.int32).reshape(NK // 128, 128)
    scores2 = top_scores.astype(jnp.float32).reshape(NK // 128, 128)

    mesh = plsc.VectorSubcoreMesh(core_axis_name="c", subcore_axis_name="s",
                                  num_cores=NC, num_subcores=NS)

    fused = pl.kernel(
        _fused_body,
        compiler_params=pltpu.CompilerParams(needs_layout_passes=False),
        out_type=(
            jax.ShapeDtypeStruct((E,), jnp.int32),
            jax.ShapeDtypeStruct((NK,), jnp.int32),
            jax.ShapeDtypeStruct((NK,), jnp.float32),
            jax.ShapeDtypeStruct((NK, D), jnp.float32),
        ),
        mesh=mesh,
        scratch_types=[
            pltpu.VMEM((RI, 128), jnp.int32),
            pltpu.VMEM((RI, 128), jnp.int32),
            pltpu.VMEM((RI, 128), jnp.int32),
            pltpu.VMEM((RI, 128), jnp.int32),
            pltpu.VMEM((RI, 128), jnp.float32),
            pltpu.VMEM((16,), jnp.int32),
            pltpu.VMEM((16,), jnp.int32),
            pltpu.VMEM((16,), jnp.int32),
            pltpu.VMEM((16,), jnp.int32),
            pltpu.VMEM((NW, 16), jnp.int32),
            pltpu.HBM((NC, NW, 16), jnp.int32),
            pltpu.VMEM_SHARED((256,), jnp.int32),
            pltpu.VMEM_SHARED((NK,), jnp.int32),
            pltpu.VMEM_SHARED((NK,), jnp.float32),
            pltpu.VMEM((C,), jnp.int32),
            pltpu.VMEM((C,), jnp.float32),
            pltpu.VMEM((RB, D), jnp.float32),
            pltpu.VMEM((RB, D), jnp.float32),
            pltpu.VMEM((RB, D), jnp.float32),
            pltpu.VMEM((RB, D), jnp.float32),
            pltpu.VMEM((RB, D), jnp.float32),
            pltpu.VMEM((RB, D), jnp.float32),
            pltpu.SemaphoreType.DMA,
            pltpu.SemaphoreType.DMA,
            pltpu.SemaphoreType.DMA,
            pltpu.SemaphoreType.DMA,
            pltpu.SemaphoreType.DMA,
            pltpu.SemaphoreType.DMA,
            pltpu.SemaphoreType.DMA,
            pltpu.SemaphoreType.DMA,
            pltpu.SemaphoreType.DMA,
            pltpu.SemaphoreType.DMA,
            pltpu.SemaphoreType.DMA,
            pltpu.SemaphoreType.DMA,
            pltpu.SemaphoreType.DMA,
            pltpu.SemaphoreType.DMA,
        ],
    )
    counts, tokidx, ssort, routed = fused(keys2, scores2, x)
    return routed, counts, tokidx, ssort
